# Initial kernel scaffold; baseline (speedup 1.0000x reference)
import jax
import jax.numpy as jnp
from jax.experimental import pallas as pl


def kernel(x, edge_index, W1l, W1r, b1, W2l, W2r, b2, W3l, W3r, b3):
    raise NotImplementedError("write your pallas kernel here")


# speedup vs baseline: 3.7493x; 3.7493x over previous
import functools

import jax
import jax.numpy as jnp
from jax import lax
from jax.experimental import pallas as pl
from jax.experimental.pallas import tpu as pltpu
from jax.experimental.pallas import tpu_sc as plsc

N_NODES = 10000
N_EDGES = 160000
SLOPE = 0.2

NC = 2
NS = 16
CH = 128
CPT = 80
E_PAD = NS * CPT * CH
N_PAD = 10112
RPT = N_PAD // NS
IDX_ROWS = E_PAD // CH

_f32 = jnp.float32


def _sc_agg_body(nchunk, tab, srcr, dstr, z128,
                 out, src_v, dst_v, rows_v, acc, sem):
    passes = nchunk // NC
    c = lax.axis_index("c")
    s = lax.axis_index("s")
    row0 = s * RPT

    pltpu.sync_copy(dstr.at[pl.ds(s * CPT, CPT)], dst_v)
    pltpu.sync_copy(z128, rows_v)

    for p in range(passes):
        n_full, rem = divmod(RPT, 128)
        for k in range(n_full):
            pltpu.sync_copy(rows_v, acc.at[pl.ds(row0 + k * 128, 128)])
        if rem:
            pltpu.sync_copy(rows_v.at[pl.ds(0, rem)],
                            acc.at[pl.ds(row0 + n_full * 128, rem)])
        plsc.subcore_barrier()

        chunk = c * passes + p
        pltpu.sync_copy(srcr.at[chunk].at[pl.ds(s * CPT, CPT)], src_v)

        @pl.loop(0, CPT)
        def _(i):
            pltpu.async_copy(tab.at[src_v.at[i]], rows_v, sem).wait()
            pltpu.sync_copy(rows_v, acc.at[dst_v.at[i]], add=True)

        plsc.subcore_barrier()
        pltpu.sync_copy(acc.at[pl.ds(row0, RPT)],
                        out.at[chunk].at[pl.ds(row0, RPT)])
        if p + 1 < passes:
            plsc.subcore_barrier()
            pltpu.sync_copy(z128, rows_v)


def _make_sc_agg(nchunk):
    mesh = plsc.VectorSubcoreMesh(core_axis_name="c", subcore_axis_name="s")
    return pl.kernel(
        functools.partial(_sc_agg_body, nchunk),
        out_type=jax.ShapeDtypeStruct((nchunk, N_PAD, 128), _f32),
        mesh=mesh,
        scratch_types=[
            pltpu.VMEM((CPT, CH), jnp.int32),
            pltpu.VMEM((CPT, CH), jnp.int32),
            pltpu.VMEM((CH, 128), _f32),
            pltpu.VMEM_SHARED((N_PAD, 128), _f32),
            pltpu.SemaphoreType.DMA,
        ],
    )


def _sc_deg_body(dstr, ones128, z128, dego, dst_v, ones_v, dega, sem):
    c = lax.axis_index("c")
    s = lax.axis_index("s")
    row0 = s * RPT
    cpt = CPT // NC

    pltpu.sync_copy(dstr.at[c].at[pl.ds(s * cpt, cpt)], dst_v)
    pltpu.sync_copy(z128, ones_v)
    n_full, rem = divmod(RPT, 128)
    for k in range(n_full):
        pltpu.sync_copy(ones_v, dega.at[pl.ds(row0 + k * 128, 128)])
    if rem:
        pltpu.sync_copy(ones_v.at[pl.ds(0, rem)],
                        dega.at[pl.ds(row0 + n_full * 128, rem)])
    pltpu.sync_copy(ones128, ones_v)
    plsc.subcore_barrier()

    @pl.loop(0, cpt)
    def _(i):
        pltpu.sync_copy(ones_v, dega.at[dst_v.at[i]], add=True)

    plsc.subcore_barrier()
    pltpu.sync_copy(dega.at[pl.ds(row0, RPT)],
                    dego.at[c].at[pl.ds(row0, RPT)])


def _make_sc_deg():
    mesh = plsc.VectorSubcoreMesh(core_axis_name="c", subcore_axis_name="s")
    return pl.kernel(
        _sc_deg_body,
        out_type=jax.ShapeDtypeStruct((NC, N_PAD, 128), _f32),
        mesh=mesh,
        scratch_types=[
            pltpu.VMEM((CPT // NC, CH), jnp.int32),
            pltpu.VMEM((CH, 128), _f32),
            pltpu.VMEM_SHARED((N_PAD, 128), _f32),
            pltpu.SemaphoreType.DMA,
        ],
    )


BN = 1000


def _split_cols(src, dst_ref, width):
    for cc in range(width // 128):
        dst_ref[cc] = src[:, cc * 128:(cc + 1) * 128]


def _tc_split_x(x):
    def body(x_ref, xs_ref):
        _split_cols(x_ref[...], xs_ref, 256)
    return pl.pallas_call(
        body,
        grid=(N_NODES // BN,),
        in_specs=[pl.BlockSpec((BN, 256), lambda i: (i, 0))],
        out_specs=pl.BlockSpec((2, BN, 128), lambda i: (0, i, 0)),
        out_shape=jax.ShapeDtypeStruct((2, N_NODES, 128), _f32),
    )(x)


def _tc_lin(h, W, b):
    n, din = h.shape
    dout = W.shape[1]
    def body(h_ref, w_ref, b_ref, z_ref):
        z_ref[...] = (
            jnp.dot(h_ref[...], w_ref[...], preferred_element_type=_f32)
            + b_ref[...])
    return pl.pallas_call(
        body,
        grid=(n // BN,),
        in_specs=[pl.BlockSpec((BN, din), lambda i: (i, 0)),
                  pl.BlockSpec((din, dout), lambda i: (0, 0)),
                  pl.BlockSpec((1, dout), lambda i: (0, 0))],
        out_specs=pl.BlockSpec((BN, dout), lambda i: (i, 0)),
        out_shape=jax.ShapeDtypeStruct((n, dout), _f32),
    )(h, W, b.reshape(1, dout))


def _mean(ag_refs, dg_ref):
    a = jnp.concatenate([r[...] for r in ag_refs], axis=1)
    deg = dg_ref[0][:, 0:1] + dg_ref[1][:, 0:1]
    r = 1.0 / jnp.maximum(deg, 1.0)
    return a * r


def _lrelu(h):
    return jnp.where(h > 0, h, SLOPE * h)


def _tc_layer1_post(aggr1, deg, z1, W1l, W2l):
    def body(ag_ref, dg_ref, z1_ref, w1l_ref, w2l_ref, y2s_ref, h1_ref):
        a = _mean([ag_ref.at[0], ag_ref.at[1]], dg_ref)
        h = _lrelu(jnp.dot(a, w1l_ref[...], preferred_element_type=_f32)
                   + z1_ref[...])
        h1_ref[...] = h
        y2 = jnp.dot(h, w2l_ref[...], preferred_element_type=_f32)
        _split_cols(y2, y2s_ref, 512)
    return pl.pallas_call(
        body,
        grid=(N_NODES // BN,),
        in_specs=[pl.BlockSpec((2, BN, 128), lambda i: (0, i, 0)),
                  pl.BlockSpec((2, BN, 128), lambda i: (0, i, 0)),
                  pl.BlockSpec((BN, 1024), lambda i: (i, 0)),
                  pl.BlockSpec((256, 1024), lambda i: (0, 0)),
                  pl.BlockSpec((1024, 512), lambda i: (0, 0))],
        out_specs=[pl.BlockSpec((4, BN, 128), lambda i: (0, i, 0)),
                   pl.BlockSpec((BN, 1024), lambda i: (i, 0))],
        out_shape=[jax.ShapeDtypeStruct((4, N_NODES, 128), _f32),
                   jax.ShapeDtypeStruct((N_NODES, 1024), _f32)],
    )(aggr1, deg, z1, W1l, W2l)


def _tc_layer2_post(aggr2, deg, z2, W3l):
    def body(ag_ref, dg_ref, z2_ref, w3l_ref, y3s_ref, h2_ref):
        a = _mean([ag_ref.at[k] for k in range(4)], dg_ref)
        h = _lrelu(a + z2_ref[...])
        h2_ref[...] = h
        y3 = jnp.dot(h, w3l_ref[...], preferred_element_type=_f32)
        _split_cols(y3, y3s_ref, 256)
    return pl.pallas_call(
        body,
        grid=(N_NODES // BN,),
        in_specs=[pl.BlockSpec((4, BN, 128), lambda i: (0, i, 0)),
                  pl.BlockSpec((2, BN, 128), lambda i: (0, i, 0)),
                  pl.BlockSpec((BN, 512), lambda i: (i, 0)),
                  pl.BlockSpec((512, 256), lambda i: (0, 0))],
        out_specs=[pl.BlockSpec((2, BN, 128), lambda i: (0, i, 0)),
                   pl.BlockSpec((BN, 512), lambda i: (i, 0))],
        out_shape=[jax.ShapeDtypeStruct((2, N_NODES, 128), _f32),
                   jax.ShapeDtypeStruct((N_NODES, 512), _f32)],
    )(aggr2, deg, z2, W3l)


def _tc_layer3_post(aggr3, deg, z3):
    def body(ag_ref, dg_ref, z3_ref, o_ref):
        o_ref[...] = _mean([ag_ref.at[0], ag_ref.at[1]], dg_ref) + z3_ref[...]
    return pl.pallas_call(
        body,
        grid=(N_NODES // BN,),
        in_specs=[pl.BlockSpec((2, BN, 128), lambda i: (0, i, 0)),
                  pl.BlockSpec((2, BN, 128), lambda i: (0, i, 0)),
                  pl.BlockSpec((BN, 256), lambda i: (i, 0))],
        out_specs=pl.BlockSpec((BN, 256), lambda i: (i, 0)),
        out_shape=jax.ShapeDtypeStruct((N_NODES, 256), _f32),
    )(aggr3, deg, z3)


@jax.jit
def kernel(x, edge_index, W1l, W1r, b1, W2l, W2r, b2, W3l, W3r, b3):
    src = edge_index[0].astype(jnp.int32)
    dst = edge_index[1].astype(jnp.int32)
    pad = E_PAD - N_EDGES
    src_p = jnp.concatenate([src, jnp.zeros((pad,), jnp.int32)])
    dst_p = jnp.concatenate(
        [dst, N_NODES + (jnp.arange(pad, dtype=jnp.int32) % 8)])
    dst_r = dst_p.reshape(IDX_ROWS, CH)
    dst_r2 = dst_p.reshape(NC, IDX_ROWS // NC, CH)

    def src_tables(nchunk):
        offs = (jnp.arange(nchunk, dtype=jnp.int32) * N_NODES)[:, None]
        return (src_p[None, :] + offs).reshape(nchunk, IDX_ROWS, CH)

    src2 = src_tables(2)
    src4 = src_tables(4)
    z128 = jnp.zeros((128, 128), _f32)
    ones128 = jnp.ones((CH, 128), _f32)

    sc_agg2 = _make_sc_agg(2)
    sc_agg4 = _make_sc_agg(4)

    deg = _make_sc_deg()(dst_r2, ones128, z128)
    xs = _tc_split_x(x)
    aggr1 = sc_agg2(xs.reshape(2 * N_NODES, 128), src2, dst_r, z128)
    z1 = _tc_lin(x, W1r, b1)
    y2s, h1 = _tc_layer1_post(aggr1, deg, z1, W1l, W2l)

    aggr2 = sc_agg4(y2s.reshape(4 * N_NODES, 128), src4, dst_r, z128)
    z2 = _tc_lin(h1, W2r, b2)
    y3s, h2 = _tc_layer2_post(aggr2, deg, z2, W3l)

    aggr3 = sc_agg2(y3s.reshape(2 * N_NODES, 128), src2, dst_r, z128)
    z3 = _tc_lin(h2, W3r, b3)
    return _tc_layer3_post(aggr3, deg, z3)


# speedup vs baseline: 4.4926x; 1.1983x over previous
import functools

import jax
import jax.numpy as jnp
from jax import lax
from jax.experimental import pallas as pl
from jax.experimental.pallas import tpu as pltpu
from jax.experimental.pallas import tpu_sc as plsc

N_NODES = 10000
N_EDGES = 160000
SLOPE = 0.2

NC = 2
NS = 16
CH = 128
CPT = 80
SEGC = 16
E_PAD = NS * CPT * CH
N_PAD = 10112
RPT = N_PAD // NS
IDX_ROWS = E_PAD // CH

_f32 = jnp.float32


def _sc_agg_body(nchunk, tab, srcr, dstr, zrow,
                 out, src_v, dst_v, rows0, rows1, acc, sem0, sem1):
    passes = nchunk // NC
    c = lax.axis_index("c")
    s = lax.axis_index("s")
    row0 = s * RPT

    def drain(buf, sem):
        pltpu.make_async_copy(tab.at[pl.ds(0, CH)], buf, sem).wait()

    for p in range(passes):
        pltpu.sync_copy(zrow, rows0)
        n_full, rem = divmod(RPT, CH)
        for k in range(n_full):
            pltpu.sync_copy(rows0, acc.at[pl.ds(row0 + k * CH, CH)])
        if rem:
            pltpu.sync_copy(rows0.at[pl.ds(0, rem)],
                            acc.at[pl.ds(row0 + n_full * CH, rem)])
        plsc.subcore_barrier()

        chunk = c * passes + p
        pltpu.sync_copy(srcr.at[chunk].at[pl.ds(s * CPT, CPT)], src_v)

        pltpu.async_copy(tab.at[src_v.at[0]], rows0, sem0)
        pltpu.async_copy(tab.at[src_v.at[1]], rows1, sem1)

        @pl.loop(0, CPT // SEGC)
        def _(g):
            base = g * SEGC
            pltpu.sync_copy(dstr.at[pl.ds(s * CPT + base, SEGC)], dst_v)

            @pl.loop(0, SEGC, step=2)
            def _(j):
                i = base + j
                drain(rows0, sem0)
                pltpu.sync_copy(rows0, acc.at[dst_v.at[j]], add=True)

                @pl.when(i + 2 < CPT)
                def _():
                    pltpu.async_copy(tab.at[src_v.at[i + 2]], rows0, sem0)

                drain(rows1, sem1)
                pltpu.sync_copy(rows1, acc.at[dst_v.at[j + 1]], add=True)

                @pl.when(i + 3 < CPT)
                def _():
                    pltpu.async_copy(tab.at[src_v.at[i + 3]], rows1, sem1)

        plsc.subcore_barrier()
        pltpu.sync_copy(acc.at[pl.ds(row0, RPT)],
                        out.at[chunk].at[pl.ds(row0, RPT)])
        if p + 1 < passes:
            plsc.subcore_barrier()


def _make_sc_agg(nchunk):
    mesh = plsc.VectorSubcoreMesh(core_axis_name="c", subcore_axis_name="s")
    return pl.kernel(
        functools.partial(_sc_agg_body, nchunk),
        out_type=jax.ShapeDtypeStruct((nchunk, N_PAD, 128), _f32),
        mesh=mesh,
        scratch_types=[
            pltpu.VMEM((CPT, CH), jnp.int32),
            pltpu.VMEM((SEGC, CH), jnp.int32),
            pltpu.VMEM((CH, 128), _f32),
            pltpu.VMEM((CH, 128), _f32),
            pltpu.VMEM_SHARED((N_PAD, 128), _f32),
            pltpu.SemaphoreType.DMA,
            pltpu.SemaphoreType.DMA,
        ],
    )


def _sc_deg_body(dstr, ones128, z128, dego, dst_v, ones_v, dega, sem):
    c = lax.axis_index("c")
    s = lax.axis_index("s")
    row0 = s * RPT
    cpt = CPT // NC

    pltpu.sync_copy(dstr.at[c].at[pl.ds(s * cpt, cpt)], dst_v)
    pltpu.sync_copy(z128, ones_v)
    n_full, rem = divmod(RPT, CH)
    for k in range(n_full):
        pltpu.sync_copy(ones_v, dega.at[pl.ds(row0 + k * CH, CH)])
    if rem:
        pltpu.sync_copy(ones_v.at[pl.ds(0, rem)],
                        dega.at[pl.ds(row0 + n_full * CH, rem)])
    pltpu.sync_copy(ones128, ones_v)
    plsc.subcore_barrier()

    @pl.loop(0, cpt)
    def _(i):
        pltpu.sync_copy(ones_v, dega.at[dst_v.at[i]], add=True)

    plsc.subcore_barrier()
    pltpu.sync_copy(dega.at[pl.ds(row0, RPT)],
                    dego.at[c].at[pl.ds(row0, RPT)])


def _make_sc_deg():
    mesh = plsc.VectorSubcoreMesh(core_axis_name="c", subcore_axis_name="s")
    return pl.kernel(
        _sc_deg_body,
        out_type=jax.ShapeDtypeStruct((NC, N_PAD, 128), _f32),
        mesh=mesh,
        scratch_types=[
            pltpu.VMEM((CPT // NC, CH), jnp.int32),
            pltpu.VMEM((CH, 128), _f32),
            pltpu.VMEM_SHARED((N_PAD, 128), _f32),
            pltpu.SemaphoreType.DMA,
        ],
    )


BN = 1000


def _split_cols(src, dst_ref, width):
    for cc in range(width // 128):
        dst_ref[cc] = src[:, cc * 128:(cc + 1) * 128]


def _tc_split_x(x):
    def body(x_ref, xs_ref):
        _split_cols(x_ref[...], xs_ref, 256)
    return pl.pallas_call(
        body,
        grid=(N_NODES // BN,),
        in_specs=[pl.BlockSpec((BN, 256), lambda i: (i, 0))],
        out_specs=pl.BlockSpec((2, BN, 128), lambda i: (0, i, 0)),
        out_shape=jax.ShapeDtypeStruct((2, N_NODES, 128), _f32),
    )(x)


def _tc_lin(h, W, b):
    n, din = h.shape
    dout = W.shape[1]
    def body(h_ref, w_ref, b_ref, z_ref):
        z_ref[...] = (
            jnp.dot(h_ref[...], w_ref[...], preferred_element_type=_f32)
            + b_ref[...])
    return pl.pallas_call(
        body,
        grid=(n // BN,),
        in_specs=[pl.BlockSpec((BN, din), lambda i: (i, 0)),
                  pl.BlockSpec((din, dout), lambda i: (0, 0)),
                  pl.BlockSpec((1, dout), lambda i: (0, 0))],
        out_specs=pl.BlockSpec((BN, dout), lambda i: (i, 0)),
        out_shape=jax.ShapeDtypeStruct((n, dout), _f32),
    )(h, W, b.reshape(1, dout))


def _mean(ag_refs, dg_ref):
    a = jnp.concatenate([r[...] for r in ag_refs], axis=1)
    deg = dg_ref[0][:, 0:1] + dg_ref[1][:, 0:1]
    r = 1.0 / jnp.maximum(deg, 1.0)
    return a * r


def _lrelu(h):
    return jnp.where(h > 0, h, SLOPE * h)


def _tc_layer1_post(aggr1, deg, z1, W1l, W2l):
    def body(ag_ref, dg_ref, z1_ref, w1l_ref, w2l_ref, y2s_ref, h1_ref):
        a = _mean([ag_ref.at[0], ag_ref.at[1]], dg_ref)
        h = _lrelu(jnp.dot(a, w1l_ref[...], preferred_element_type=_f32)
                   + z1_ref[...])
        h1_ref[...] = h
        y2 = jnp.dot(h, w2l_ref[...], preferred_element_type=_f32)
        _split_cols(y2, y2s_ref, 512)
    return pl.pallas_call(
        body,
        grid=(N_NODES // BN,),
        in_specs=[pl.BlockSpec((2, BN, 128), lambda i: (0, i, 0)),
                  pl.BlockSpec((2, BN, 128), lambda i: (0, i, 0)),
                  pl.BlockSpec((BN, 1024), lambda i: (i, 0)),
                  pl.BlockSpec((256, 1024), lambda i: (0, 0)),
                  pl.BlockSpec((1024, 512), lambda i: (0, 0))],
        out_specs=[pl.BlockSpec((4, BN, 128), lambda i: (0, i, 0)),
                   pl.BlockSpec((BN, 1024), lambda i: (i, 0))],
        out_shape=[jax.ShapeDtypeStruct((4, N_NODES, 128), _f32),
                   jax.ShapeDtypeStruct((N_NODES, 1024), _f32)],
    )(aggr1, deg, z1, W1l, W2l)


def _tc_layer2_post(aggr2, deg, z2, W3l):
    def body(ag_ref, dg_ref, z2_ref, w3l_ref, y3s_ref, h2_ref):
        a = _mean([ag_ref.at[k] for k in range(4)], dg_ref)
        h = _lrelu(a + z2_ref[...])
        h2_ref[...] = h
        y3 = jnp.dot(h, w3l_ref[...], preferred_element_type=_f32)
        _split_cols(y3, y3s_ref, 256)
    return pl.pallas_call(
        body,
        grid=(N_NODES // BN,),
        in_specs=[pl.BlockSpec((4, BN, 128), lambda i: (0, i, 0)),
                  pl.BlockSpec((2, BN, 128), lambda i: (0, i, 0)),
                  pl.BlockSpec((BN, 512), lambda i: (i, 0)),
                  pl.BlockSpec((512, 256), lambda i: (0, 0))],
        out_specs=[pl.BlockSpec((2, BN, 128), lambda i: (0, i, 0)),
                   pl.BlockSpec((BN, 512), lambda i: (i, 0))],
        out_shape=[jax.ShapeDtypeStruct((2, N_NODES, 128), _f32),
                   jax.ShapeDtypeStruct((N_NODES, 512), _f32)],
    )(aggr2, deg, z2, W3l)


def _tc_layer3_post(aggr3, deg, z3):
    def body(ag_ref, dg_ref, z3_ref, o_ref):
        o_ref[...] = _mean([ag_ref.at[0], ag_ref.at[1]], dg_ref) + z3_ref[...]
    return pl.pallas_call(
        body,
        grid=(N_NODES // BN,),
        in_specs=[pl.BlockSpec((2, BN, 128), lambda i: (0, i, 0)),
                  pl.BlockSpec((2, BN, 128), lambda i: (0, i, 0)),
                  pl.BlockSpec((BN, 256), lambda i: (i, 0))],
        out_specs=pl.BlockSpec((BN, 256), lambda i: (i, 0)),
        out_shape=jax.ShapeDtypeStruct((N_NODES, 256), _f32),
    )(aggr3, deg, z3)


@jax.jit
def kernel(x, edge_index, W1l, W1r, b1, W2l, W2r, b2, W3l, W3r, b3):
    src = edge_index[0].astype(jnp.int32)
    dst = edge_index[1].astype(jnp.int32)
    pad = E_PAD - N_EDGES
    src_p = jnp.concatenate([src, jnp.zeros((pad,), jnp.int32)])
    dst_p = jnp.concatenate(
        [dst, N_NODES + (jnp.arange(pad, dtype=jnp.int32) % 8)])
    dst_r = dst_p.reshape(IDX_ROWS, CH)
    dst_r2 = dst_p.reshape(NC, IDX_ROWS // NC, CH)

    def src_tables(nchunk):
        offs = (jnp.arange(nchunk, dtype=jnp.int32) * N_NODES)[:, None]
        return (src_p[None, :] + offs).reshape(nchunk, IDX_ROWS, CH)

    src2 = src_tables(2)
    src4 = src_tables(4)
    zrow = jnp.zeros((CH, 128), _f32)
    ones_b = jnp.ones((CH, 128), _f32)

    sc_agg2 = _make_sc_agg(2)
    sc_agg4 = _make_sc_agg(4)

    deg = _make_sc_deg()(dst_r2, ones_b, zrow)
    xs = _tc_split_x(x)
    aggr1 = sc_agg2(xs.reshape(2 * N_NODES, 128), src2, dst_r, zrow)
    z1 = _tc_lin(x, W1r, b1)
    y2s, h1 = _tc_layer1_post(aggr1, deg, z1, W1l, W2l)

    aggr2 = sc_agg4(y2s.reshape(4 * N_NODES, 128), src4, dst_r, zrow)
    z2 = _tc_lin(h1, W2r, b2)
    y3s, h2 = _tc_layer2_post(aggr2, deg, z2, W3l)

    aggr3 = sc_agg2(y3s.reshape(2 * N_NODES, 128), src2, dst_r, zrow)
    z3 = _tc_lin(h2, W3r, b3)
    return _tc_layer3_post(aggr3, deg, z3)
